# baseline (speedup 1.0000x reference)
import functools

import jax
import jax.numpy as jnp
from jax import lax
from jax.experimental import pallas as pl
from jax.experimental.pallas import tpu as pltpu

_IMAGE_W = 1024
_IMAGE_H = 1024
_MIN_SCORE = 0.1
_MAX_DET = 50
_A_PAD = 20480
_BLK = 2048
_NEG = float("-inf")


def _trunc(x):
    return jnp.where(x >= 0.0, jnp.floor(x), jnp.ceil(x))


def _prep_body(cls_ref, reg_ref, anc_ref,
               s_ref, c_ref, x1_ref, y1_ref, x2_ref, y2_ref, ar_ref):
    c = cls_ref[0]
    s = jnp.max(c, axis=0, keepdims=True)
    iota_c = lax.broadcasted_iota(jnp.int32, c.shape, 0)
    cls_i = jnp.min(jnp.where(c == s, iota_c, c.shape[0]),
                    axis=0, keepdims=True)
    reg = reg_ref[0]
    anc = anc_ref[0]
    aw = anc[2:3] - anc[0:1]
    ah = anc[3:4] - anc[1:2]
    acx = anc[0:1] + 0.5 * aw
    acy = anc[1:2] + 0.5 * ah
    tx = reg[0:1] * 0.1
    ty = reg[1:2] * 0.1
    tw = reg[2:3] * 0.2
    th = reg[3:4] * 0.2
    w = jnp.exp(tw) * aw
    h = jnp.exp(th) * ah
    cx = tx * aw + acx
    cy = ty * ah + acy
    x1 = jnp.maximum(_trunc(cx - 0.5 * w), 0.0)
    y1 = jnp.maximum(_trunc(cy - 0.5 * h), 0.0)
    x2 = jnp.minimum(_trunc(cx + 0.5 * w), float(_IMAGE_W - 1))
    y2 = jnp.minimum(_trunc(cy + 0.5 * h), float(_IMAGE_H - 1))
    area = jnp.maximum(x2 - x1, 0.0) * jnp.maximum(y2 - y1, 0.0)
    s_ref[0, 0] = jnp.where(s > _MIN_SCORE, s, _NEG)
    c_ref[0, 0] = cls_i.astype(jnp.float32)
    x1_ref[0, 0] = x1
    y1_ref[0, 0] = y1
    x2_ref[0, 0] = x2
    y2_ref[0, 0] = y2
    ar_ref[0, 0] = area


def _nms_body(s_ref, c_ref, x1_ref, y1_ref, x2_ref, y2_ref, ar_ref,
              os_ref, oc_ref, ox1_ref, oy1_ref, ox2_ref, oy2_ref,
              sw_ref, rm_ref, kar_ref):
    B, R, L = s_ref.shape
    sw_ref[...] = s_ref[...]
    rm_ref[...] = jnp.max(s_ref[...], axis=2)
    fill = jnp.full((B, 64), -1.0, jnp.float32)
    os_ref[...] = fill
    oc_ref[...] = fill
    ox1_ref[...] = fill
    oy1_ref[...] = fill
    ox2_ref[...] = fill
    oy2_ref[...] = fill
    kar_ref[...] = jnp.zeros((B, 64), jnp.float32)
    iotaR = lax.broadcasted_iota(jnp.int32, (1, R), 1)
    lane128 = lax.broadcasted_iota(jnp.int32, (1, L), 1)
    lane64 = lax.broadcasted_iota(jnp.int32, (1, 64), 1)

    def body(carry):
        acts, kcnts = carry
        new_acts, new_kcnts = [], []
        for b in range(B):
            act, kcnt = acts[b], kcnts[b]
            rm = rm_ref[pl.ds(b, 1), :]
            m = jnp.max(rm)
            found = m > _NEG
            proceed = act & found
            r = jnp.minimum(jnp.min(jnp.where(rm == m, iotaR, R)), R - 1)
            row = sw_ref[b, pl.ds(r, 1), :]
            pc = jnp.minimum(jnp.min(jnp.where(row == m, lane128, L)), L - 1)
            oh128 = lane128 == pc

            def gat(ref):
                return jnp.sum(jnp.where(oh128, ref[b, pl.ds(r, 1), :], 0.0))

            px1 = gat(x1_ref)
            py1 = gat(y1_ref)
            px2 = gat(x2_ref)
            py2 = gat(y2_ref)
            pcl = gat(c_ref)
            par = gat(ar_ref)

            newrow = jnp.where(oh128, _NEG, row)
            sw_ref[b, pl.ds(r, 1), :] = jnp.where(proceed, newrow, row)
            rm_ref[pl.ds(b, 1), :] = jnp.where(
                proceed & (iotaR == r), jnp.max(newrow), rm)

            kx1 = ox1_ref[pl.ds(b, 1), :]
            ky1 = oy1_ref[pl.ds(b, 1), :]
            kx2 = ox2_ref[pl.ds(b, 1), :]
            ky2 = oy2_ref[pl.ds(b, 1), :]
            kcl = oc_ref[pl.ds(b, 1), :]
            kar = kar_ref[pl.ds(b, 1), :]
            iw = jnp.maximum(jnp.minimum(px2, kx2) - jnp.maximum(px1, kx1), 0.0)
            ih = jnp.maximum(jnp.minimum(py2, ky2) - jnp.maximum(py1, ky1), 0.0)
            inter = iw * ih
            bad = ((lane64 < kcnt) & (kcl == pcl)
                   & (inter > 0.5 * (par + kar - inter + 1e-9)))
            accept = proceed & jnp.logical_not(jnp.any(bad))

            wh = accept & (lane64 == kcnt)
            os_ref[pl.ds(b, 1), :] = jnp.where(wh, m, os_ref[pl.ds(b, 1), :])
            oc_ref[pl.ds(b, 1), :] = jnp.where(wh, pcl, kcl)
            ox1_ref[pl.ds(b, 1), :] = jnp.where(wh, px1, kx1)
            oy1_ref[pl.ds(b, 1), :] = jnp.where(wh, py1, ky1)
            ox2_ref[pl.ds(b, 1), :] = jnp.where(wh, px2, kx2)
            oy2_ref[pl.ds(b, 1), :] = jnp.where(wh, py2, ky2)
            kar_ref[pl.ds(b, 1), :] = jnp.where(wh, par, kar)

            kcnt = kcnt + accept.astype(jnp.int32)
            new_kcnts.append(kcnt)
            new_acts.append(proceed & (kcnt < _MAX_DET))
        return tuple(new_acts), tuple(new_kcnts)

    def cond(carry):
        acts, _ = carry
        out = acts[0]
        for a in acts[1:]:
            out = out | a
        return out

    init = (tuple(jnp.bool_(True) for _ in range(B)),
            tuple(jnp.int32(0) for _ in range(B)))
    lax.while_loop(cond, body, init)


@jax.jit
def kernel(cls_heads, reg_heads, batch_anchors):
    B, A, C = cls_heads.shape
    pad = _A_PAD - A
    cls_t = jnp.pad(jnp.transpose(cls_heads, (0, 2, 1)),
                    ((0, 0), (0, 0), (0, pad)), constant_values=-1e30)
    reg_t = jnp.pad(jnp.transpose(reg_heads, (0, 2, 1)),
                    ((0, 0), (0, 0), (0, pad)))
    anc_t = jnp.pad(jnp.transpose(batch_anchors, (0, 2, 1)),
                    ((0, 0), (0, 0), (0, pad)))

    nblk = _A_PAD // _BLK
    flat = jax.ShapeDtypeStruct((B, nblk, 1, _BLK), jnp.float32)
    prep = pl.pallas_call(
        _prep_body,
        grid=(B, nblk),
        in_specs=[
            pl.BlockSpec((1, C, _BLK), lambda b, j: (b, 0, j)),
            pl.BlockSpec((1, 4, _BLK), lambda b, j: (b, 0, j)),
            pl.BlockSpec((1, 4, _BLK), lambda b, j: (b, 0, j)),
        ],
        out_specs=[pl.BlockSpec((1, 1, 1, _BLK), lambda b, j: (b, j, 0, 0))] * 7,
        out_shape=[flat] * 7,
    )
    s_w, cls_f, x1, y1, x2, y2, area = prep(cls_t, reg_t, anc_t)

    rs = lambda a: a.reshape(B, _A_PAD // 128, 128)
    out64 = jax.ShapeDtypeStruct((B, 64), jnp.float32)
    nms = pl.pallas_call(
        _nms_body,
        out_shape=[out64] * 6,
        scratch_shapes=[pltpu.VMEM((B, _A_PAD // 128, 128), jnp.float32),
                        pltpu.VMEM((B, _A_PAD // 128), jnp.float32),
                        pltpu.VMEM((B, 64), jnp.float32)],
    )
    ss, cc, bx1, by1, bx2, by2 = nms(
        rs(s_w), rs(cls_f), rs(x1), rs(y1), rs(x2), rs(y2), rs(area))

    boxes = jnp.stack([bx1[:, :_MAX_DET], by1[:, :_MAX_DET],
                       bx2[:, :_MAX_DET], by2[:, :_MAX_DET]], axis=-1)
    return ss[:, :_MAX_DET], cc[:, :_MAX_DET], boxes


# speedup vs baseline: 1.3742x; 1.3742x over previous
import functools

import jax
import jax.numpy as jnp
from jax import lax
from jax.experimental import pallas as pl
from jax.experimental.pallas import tpu as pltpu

_IMAGE_W = 1024
_IMAGE_H = 1024
_MIN_SCORE = 0.1
_MAX_DET = 50
_A_PAD = 20480
_BLK = 2048
_NEG = float("-inf")


def _trunc(x):
    return jnp.where(x >= 0.0, jnp.floor(x), jnp.ceil(x))


def _prep_body(cls_ref, reg_ref, anc_ref,
               s_ref, c_ref, x1_ref, y1_ref, x2_ref, y2_ref, ar_ref):
    c = cls_ref[0]
    s = jnp.max(c, axis=0, keepdims=True)
    iota_c = lax.broadcasted_iota(jnp.int32, c.shape, 0)
    cls_i = jnp.min(jnp.where(c == s, iota_c, c.shape[0]),
                    axis=0, keepdims=True)
    reg = reg_ref[0]
    anc = anc_ref[0]
    aw = anc[2:3] - anc[0:1]
    ah = anc[3:4] - anc[1:2]
    acx = anc[0:1] + 0.5 * aw
    acy = anc[1:2] + 0.5 * ah
    tx = reg[0:1] * 0.1
    ty = reg[1:2] * 0.1
    tw = reg[2:3] * 0.2
    th = reg[3:4] * 0.2
    w = jnp.exp(tw) * aw
    h = jnp.exp(th) * ah
    cx = tx * aw + acx
    cy = ty * ah + acy
    x1 = jnp.maximum(_trunc(cx - 0.5 * w), 0.0)
    y1 = jnp.maximum(_trunc(cy - 0.5 * h), 0.0)
    x2 = jnp.minimum(_trunc(cx + 0.5 * w), float(_IMAGE_W - 1))
    y2 = jnp.minimum(_trunc(cy + 0.5 * h), float(_IMAGE_H - 1))
    area = jnp.maximum(x2 - x1, 0.0) * jnp.maximum(y2 - y1, 0.0)
    s_ref[0, 0] = jnp.where(s > _MIN_SCORE, s, _NEG)
    c_ref[0, 0] = cls_i.astype(jnp.float32)
    x1_ref[0, 0] = x1
    y1_ref[0, 0] = y1
    x2_ref[0, 0] = x2
    y2_ref[0, 0] = y2
    ar_ref[0, 0] = area


def _nms_body(s_ref, c_ref, x1_ref, y1_ref, x2_ref, y2_ref, ar_ref,
              os_ref, oc_ref, ox1_ref, oy1_ref, ox2_ref, oy2_ref,
              sw_ref):
    B, R, L = s_ref.shape
    sw_ref[...] = s_ref[...]
    fill = jnp.full((B, 64), -1.0, jnp.float32)
    os_ref[...] = fill
    oc_ref[...] = fill
    ox1_ref[...] = fill
    oy1_ref[...] = fill
    ox2_ref[...] = fill
    oy2_ref[...] = fill
    lin = (lax.broadcasted_iota(jnp.int32, (R, L), 0) * L
           + lax.broadcasted_iota(jnp.int32, (R, L), 1))
    lane128 = lax.broadcasted_iota(jnp.int32, (1, L), 1)
    lane64 = lax.broadcasted_iota(jnp.int32, (1, 64), 1)

    def body(t, _):
        for b in range(B):
            sb = sw_ref[b]
            m = jnp.max(sb)
            has = m > _NEG
            pick = jnp.min(jnp.where(sb == m, lin, R * L))
            pr = jnp.minimum(pick // L, R - 1)
            pc = pick % L

            def gat(ref):
                row = ref[b, pl.ds(pr, 1), :]
                return jnp.sum(jnp.where(lane128 == pc, row, 0.0))

            px1 = gat(x1_ref)
            py1 = gat(y1_ref)
            px2 = gat(x2_ref)
            py2 = gat(y2_ref)
            pcl = gat(c_ref)
            par = gat(ar_ref)

            iw = jnp.maximum(jnp.minimum(px2, x2_ref[b])
                             - jnp.maximum(px1, x1_ref[b]), 0.0)
            ih = jnp.maximum(jnp.minimum(py2, y2_ref[b])
                             - jnp.maximum(py1, y1_ref[b]), 0.0)
            inter = iw * ih
            thr = 0.5 * (par + ar_ref[b] - inter + 1e-9)
            sup = (inter > thr) & (c_ref[b] == pcl)
            kill = (sup | (lin == pick)) & has
            sw_ref[b] = jnp.where(kill, _NEG, sb)

            oh = (lane64 == t) & has
            row_s = os_ref[pl.ds(b, 1), :]
            os_ref[pl.ds(b, 1), :] = jnp.where(oh, m, row_s)
            oc_ref[pl.ds(b, 1), :] = jnp.where(oh, pcl, oc_ref[pl.ds(b, 1), :])
            ox1_ref[pl.ds(b, 1), :] = jnp.where(oh, px1, ox1_ref[pl.ds(b, 1), :])
            oy1_ref[pl.ds(b, 1), :] = jnp.where(oh, py1, oy1_ref[pl.ds(b, 1), :])
            ox2_ref[pl.ds(b, 1), :] = jnp.where(oh, px2, ox2_ref[pl.ds(b, 1), :])
            oy2_ref[pl.ds(b, 1), :] = jnp.where(oh, py2, oy2_ref[pl.ds(b, 1), :])
        return 0

    lax.fori_loop(0, _MAX_DET, body, 0)


@jax.jit
def kernel(cls_heads, reg_heads, batch_anchors):
    B, A, C = cls_heads.shape
    pad = _A_PAD - A
    cls_t = jnp.pad(jnp.transpose(cls_heads, (0, 2, 1)),
                    ((0, 0), (0, 0), (0, pad)), constant_values=-1e30)
    reg_t = jnp.pad(jnp.transpose(reg_heads, (0, 2, 1)),
                    ((0, 0), (0, 0), (0, pad)))
    anc_t = jnp.pad(jnp.transpose(batch_anchors, (0, 2, 1)),
                    ((0, 0), (0, 0), (0, pad)))

    nblk = _A_PAD // _BLK
    flat = jax.ShapeDtypeStruct((B, nblk, 1, _BLK), jnp.float32)
    prep = pl.pallas_call(
        _prep_body,
        grid=(B, nblk),
        in_specs=[
            pl.BlockSpec((1, C, _BLK), lambda b, j: (b, 0, j)),
            pl.BlockSpec((1, 4, _BLK), lambda b, j: (b, 0, j)),
            pl.BlockSpec((1, 4, _BLK), lambda b, j: (b, 0, j)),
        ],
        out_specs=[pl.BlockSpec((1, 1, 1, _BLK), lambda b, j: (b, j, 0, 0))] * 7,
        out_shape=[flat] * 7,
    )
    s_w, cls_f, x1, y1, x2, y2, area = prep(cls_t, reg_t, anc_t)

    rs = lambda a: a.reshape(B, _A_PAD // 128, 128)
    out64 = jax.ShapeDtypeStruct((B, 64), jnp.float32)
    nms = pl.pallas_call(
        _nms_body,
        out_shape=[out64] * 6,
        scratch_shapes=[pltpu.VMEM((B, _A_PAD // 128, 128), jnp.float32)],
    )
    ss, cc, bx1, by1, bx2, by2 = nms(
        rs(s_w), rs(cls_f), rs(x1), rs(y1), rs(x2), rs(y2), rs(area))

    boxes = jnp.stack([bx1[:, :_MAX_DET], by1[:, :_MAX_DET],
                       bx2[:, :_MAX_DET], by2[:, :_MAX_DET]], axis=-1)
    return ss[:, :_MAX_DET], cc[:, :_MAX_DET], boxes


# speedup vs baseline: 1.6120x; 1.1730x over previous
import functools

import jax
import jax.numpy as jnp
from jax import lax
from jax.experimental import pallas as pl
from jax.experimental.pallas import tpu as pltpu

_IMAGE_W = 1024
_IMAGE_H = 1024
_MIN_SCORE = 0.1
_MAX_DET = 50
_A = 20000
_A_PAD = 20480
_BLK = 2048
_NEG = float("-inf")


def _trunc(x):
    return jnp.where(x >= 0.0, jnp.floor(x), jnp.ceil(x))


def _prep_body(cls_ref, reg_ref, anc_ref,
               s_ref, c_ref, x1_ref, y1_ref, x2_ref, y2_ref):
    j = pl.program_id(1)
    lane = lax.broadcasted_iota(jnp.int32, (1, _BLK), 1)
    valid = (j * _BLK + lane) < _A
    c = cls_ref[0]
    s = jnp.max(c, axis=0, keepdims=True)
    iota_c = lax.broadcasted_iota(jnp.int32, c.shape, 0)
    cls_i = jnp.min(jnp.where(c == s, iota_c, c.shape[0]),
                    axis=0, keepdims=True)
    reg = reg_ref[0]
    anc = anc_ref[0]
    aw = anc[2:3] - anc[0:1]
    ah = anc[3:4] - anc[1:2]
    acx = anc[0:1] + 0.5 * aw
    acy = anc[1:2] + 0.5 * ah
    tx = reg[0:1] * 0.1
    ty = reg[1:2] * 0.1
    tw = reg[2:3] * 0.2
    th = reg[3:4] * 0.2
    w = jnp.exp(tw) * aw
    h = jnp.exp(th) * ah
    cx = tx * aw + acx
    cy = ty * ah + acy
    x1 = jnp.maximum(_trunc(cx - 0.5 * w), 0.0)
    y1 = jnp.maximum(_trunc(cy - 0.5 * h), 0.0)
    x2 = jnp.minimum(_trunc(cx + 0.5 * w), float(_IMAGE_W - 1))
    y2 = jnp.minimum(_trunc(cy + 0.5 * h), float(_IMAGE_H - 1))
    s_ref[0, 0] = jnp.where(valid & (s > _MIN_SCORE), s, _NEG)
    c_ref[0, 0] = jnp.where(valid, cls_i.astype(jnp.float32), 0.0)
    x1_ref[0, 0] = jnp.where(valid, x1, 0.0)
    y1_ref[0, 0] = jnp.where(valid, y1, 0.0)
    x2_ref[0, 0] = jnp.where(valid, x2, 0.0)
    y2_ref[0, 0] = jnp.where(valid, y2, 0.0)


def _nms_body(s_ref, c_ref, x1_ref, y1_ref, x2_ref, y2_ref,
              os_ref, oc_ref, ox1_ref, oy1_ref, ox2_ref, oy2_ref,
              sw_ref):
    B, R, L = s_ref.shape
    sw_ref[...] = s_ref[...]
    fill = jnp.full((B, 64), -1.0, jnp.float32)
    os_ref[...] = fill
    oc_ref[...] = fill
    ox1_ref[...] = fill
    oy1_ref[...] = fill
    ox2_ref[...] = fill
    oy2_ref[...] = fill
    lin = (lax.broadcasted_iota(jnp.int32, (R, L), 0) * L
           + lax.broadcasted_iota(jnp.int32, (R, L), 1))
    lane128 = lax.broadcasted_iota(jnp.int32, (1, L), 1)
    lane64 = lax.broadcasted_iota(jnp.int32, (1, 64), 1)

    def body(t, _):
        for b in range(B):
            sb = sw_ref[b]
            m = jnp.max(sb)
            has = m > _NEG
            pick = jnp.min(jnp.where(sb == m, lin, R * L))
            pr = jnp.minimum(pick // L, R - 1)
            pc = pick % L

            def gat(ref):
                row = ref[b, pl.ds(pr, 1), :]
                return jnp.sum(jnp.where(lane128 == pc, row, 0.0))

            px1 = gat(x1_ref)
            py1 = gat(y1_ref)
            px2 = gat(x2_ref)
            py2 = gat(y2_ref)
            pcl = gat(c_ref)
            par = (jnp.maximum(px2 - px1, 0.0)
                   * jnp.maximum(py2 - py1, 0.0))

            X1 = x1_ref[b]
            Y1 = y1_ref[b]
            X2 = x2_ref[b]
            Y2 = y2_ref[b]
            areas = jnp.maximum(X2 - X1, 0.0) * jnp.maximum(Y2 - Y1, 0.0)
            iw = jnp.maximum(jnp.minimum(px2, X2) - jnp.maximum(px1, X1), 0.0)
            ih = jnp.maximum(jnp.minimum(py2, Y2) - jnp.maximum(py1, Y1), 0.0)
            inter = iw * ih
            thr = 0.5 * (par + areas - inter + 1e-9)
            sup = (inter > thr) & (c_ref[b] == pcl)
            kill = (sup | (lin == pick)) & has
            sw_ref[b] = jnp.where(kill, _NEG, sb)

            oh = (lane64 == t) & has
            row_s = os_ref[pl.ds(b, 1), :]
            os_ref[pl.ds(b, 1), :] = jnp.where(oh, m, row_s)
            oc_ref[pl.ds(b, 1), :] = jnp.where(oh, pcl, oc_ref[pl.ds(b, 1), :])
            ox1_ref[pl.ds(b, 1), :] = jnp.where(oh, px1, ox1_ref[pl.ds(b, 1), :])
            oy1_ref[pl.ds(b, 1), :] = jnp.where(oh, py1, oy1_ref[pl.ds(b, 1), :])
            ox2_ref[pl.ds(b, 1), :] = jnp.where(oh, px2, ox2_ref[pl.ds(b, 1), :])
            oy2_ref[pl.ds(b, 1), :] = jnp.where(oh, py2, oy2_ref[pl.ds(b, 1), :])
        return 0

    lax.fori_loop(0, _MAX_DET, body, 0)


@jax.jit
def kernel(cls_heads, reg_heads, batch_anchors):
    B, A, C = cls_heads.shape
    cls_t = jnp.transpose(cls_heads, (0, 2, 1))
    reg_t = jnp.transpose(reg_heads, (0, 2, 1))
    anc_t = jnp.transpose(batch_anchors, (0, 2, 1))

    nblk = _A_PAD // _BLK
    flat = jax.ShapeDtypeStruct((B, nblk, 1, _BLK), jnp.float32)
    prep = pl.pallas_call(
        _prep_body,
        grid=(B, nblk),
        in_specs=[
            pl.BlockSpec((1, C, _BLK), lambda b, j: (b, 0, j)),
            pl.BlockSpec((1, 4, _BLK), lambda b, j: (b, 0, j)),
            pl.BlockSpec((1, 4, _BLK), lambda b, j: (b, 0, j)),
        ],
        out_specs=[pl.BlockSpec((1, 1, 1, _BLK), lambda b, j: (b, j, 0, 0))] * 6,
        out_shape=[flat] * 6,
    )
    s_w, cls_f, x1, y1, x2, y2 = prep(cls_t, reg_t, anc_t)

    rs = lambda a: a.reshape(B, _A_PAD // 128, 128)
    out64 = jax.ShapeDtypeStruct((B, 64), jnp.float32)
    nms = pl.pallas_call(
        _nms_body,
        out_shape=[out64] * 6,
        scratch_shapes=[pltpu.VMEM((B, _A_PAD // 128, 128), jnp.float32)],
    )
    ss, cc, bx1, by1, bx2, by2 = nms(
        rs(s_w), rs(cls_f), rs(x1), rs(y1), rs(x2), rs(y2))

    boxes = jnp.stack([bx1[:, :_MAX_DET], by1[:, :_MAX_DET],
                       bx2[:, :_MAX_DET], by2[:, :_MAX_DET]], axis=-1)
    return ss[:, :_MAX_DET], cc[:, :_MAX_DET], boxes


# speedup vs baseline: 1.6122x; 1.0001x over previous
import functools

import jax
import jax.numpy as jnp
from jax import lax
from jax.experimental import pallas as pl
from jax.experimental.pallas import tpu as pltpu

_IMAGE_W = 1024
_IMAGE_H = 1024
_MIN_SCORE = 0.1
_MAX_DET = 50
_A = 20000
_A_PAD = 20480
_BLK = 2048
_NEG = float("-inf")


def _trunc(x):
    return jnp.where(x >= 0.0, jnp.floor(x), jnp.ceil(x))


def _prep_body(cls_ref, reg_ref, anc_ref,
               s_ref, c_ref, x1_ref, y1_ref, x2_ref, y2_ref):
    j = pl.program_id(1)
    lane = lax.broadcasted_iota(jnp.int32, (1, _BLK), 1)
    valid = (j * _BLK + lane) < _A
    c = cls_ref[0]
    s = jnp.max(c, axis=0, keepdims=True)
    iota_c = lax.broadcasted_iota(jnp.int32, c.shape, 0)
    cls_i = jnp.min(jnp.where(c == s, iota_c, c.shape[0]),
                    axis=0, keepdims=True)
    reg = reg_ref[0]
    anc = anc_ref[0]
    aw = anc[2:3] - anc[0:1]
    ah = anc[3:4] - anc[1:2]
    acx = anc[0:1] + 0.5 * aw
    acy = anc[1:2] + 0.5 * ah
    tx = reg[0:1] * 0.1
    ty = reg[1:2] * 0.1
    tw = reg[2:3] * 0.2
    th = reg[3:4] * 0.2
    w = jnp.exp(tw) * aw
    h = jnp.exp(th) * ah
    cx = tx * aw + acx
    cy = ty * ah + acy
    x1 = jnp.maximum(_trunc(cx - 0.5 * w), 0.0)
    y1 = jnp.maximum(_trunc(cy - 0.5 * h), 0.0)
    x2 = jnp.minimum(_trunc(cx + 0.5 * w), float(_IMAGE_W - 1))
    y2 = jnp.minimum(_trunc(cy + 0.5 * h), float(_IMAGE_H - 1))
    cls_f = cls_i.astype(jnp.float32)
    off = cls_f * float(_IMAGE_W)
    s_ref[0, 0] = jnp.where(valid & (s > _MIN_SCORE), s, _NEG)
    c_ref[0, 0] = jnp.where(valid, cls_f, 0.0)
    x1_ref[0, 0] = jnp.where(valid, x1 + off, 0.0)
    y1_ref[0, 0] = jnp.where(valid, y1, 0.0)
    x2_ref[0, 0] = jnp.where(valid, x2 + off, 0.0)
    y2_ref[0, 0] = jnp.where(valid, y2, 0.0)


def _nms_body(s_ref, c_ref, x1_ref, y1_ref, x2_ref, y2_ref,
              os_ref, oc_ref, ox1_ref, oy1_ref, ox2_ref, oy2_ref,
              sw_ref):
    B, R, L = s_ref.shape
    sw_ref[...] = s_ref[...]
    fill = jnp.full((B, 64), -1.0, jnp.float32)
    os_ref[...] = fill
    oc_ref[...] = fill
    ox1_ref[...] = fill
    oy1_ref[...] = fill
    ox2_ref[...] = fill
    oy2_ref[...] = fill
    lin = (lax.broadcasted_iota(jnp.int32, (R, L), 0) * L
           + lax.broadcasted_iota(jnp.int32, (R, L), 1))
    lane128 = lax.broadcasted_iota(jnp.int32, (1, L), 1)
    lane64 = lax.broadcasted_iota(jnp.int32, (1, 64), 1)

    def body(t, _):
        for b in range(B):
            sb = sw_ref[b]
            m = jnp.max(sb)
            has = m > _NEG
            pick = jnp.min(jnp.where(sb == m, lin, R * L))
            pr = jnp.minimum(pick // L, R - 1)
            pc = pick % L

            def gat(ref):
                row = ref[b, pl.ds(pr, 1), :]
                return jnp.sum(jnp.where(lane128 == pc, row, 0.0))

            px1 = gat(x1_ref)
            py1 = gat(y1_ref)
            px2 = gat(x2_ref)
            py2 = gat(y2_ref)
            pcl = gat(c_ref)
            par = (jnp.maximum(px2 - px1, 0.0)
                   * jnp.maximum(py2 - py1, 0.0))

            X1 = x1_ref[b]
            Y1 = y1_ref[b]
            X2 = x2_ref[b]
            Y2 = y2_ref[b]
            areas = jnp.maximum(X2 - X1, 0.0) * jnp.maximum(Y2 - Y1, 0.0)
            iw = jnp.maximum(jnp.minimum(px2, X2) - jnp.maximum(px1, X1), 0.0)
            ih = jnp.maximum(jnp.minimum(py2, Y2) - jnp.maximum(py1, Y1), 0.0)
            inter = iw * ih
            thr = 0.5 * (par + areas - inter + 1e-9)
            kill = ((inter > thr) | (lin == pick)) & has
            sw_ref[b] = jnp.where(kill, _NEG, sb)

            poff = pcl * float(_IMAGE_W)
            oh = (lane64 == t) & has
            row_s = os_ref[pl.ds(b, 1), :]
            os_ref[pl.ds(b, 1), :] = jnp.where(oh, m, row_s)
            oc_ref[pl.ds(b, 1), :] = jnp.where(oh, pcl, oc_ref[pl.ds(b, 1), :])
            ox1_ref[pl.ds(b, 1), :] = jnp.where(oh, px1 - poff, ox1_ref[pl.ds(b, 1), :])
            oy1_ref[pl.ds(b, 1), :] = jnp.where(oh, py1, oy1_ref[pl.ds(b, 1), :])
            ox2_ref[pl.ds(b, 1), :] = jnp.where(oh, px2 - poff, ox2_ref[pl.ds(b, 1), :])
            oy2_ref[pl.ds(b, 1), :] = jnp.where(oh, py2, oy2_ref[pl.ds(b, 1), :])
        return 0

    lax.fori_loop(0, _MAX_DET, body, 0)


@jax.jit
def kernel(cls_heads, reg_heads, batch_anchors):
    B, A, C = cls_heads.shape
    cls_t = jnp.transpose(cls_heads, (0, 2, 1))
    reg_t = jnp.transpose(reg_heads, (0, 2, 1))
    anc_t = jnp.transpose(batch_anchors, (0, 2, 1))

    nblk = _A_PAD // _BLK
    flat = jax.ShapeDtypeStruct((B, nblk, 1, _BLK), jnp.float32)
    prep = pl.pallas_call(
        _prep_body,
        grid=(B, nblk),
        in_specs=[
            pl.BlockSpec((1, C, _BLK), lambda b, j: (b, 0, j)),
            pl.BlockSpec((1, 4, _BLK), lambda b, j: (b, 0, j)),
            pl.BlockSpec((1, 4, _BLK), lambda b, j: (b, 0, j)),
        ],
        out_specs=[pl.BlockSpec((1, 1, 1, _BLK), lambda b, j: (b, j, 0, 0))] * 6,
        out_shape=[flat] * 6,
    )
    s_w, cls_f, x1, y1, x2, y2 = prep(cls_t, reg_t, anc_t)

    rs = lambda a: a.reshape(B, _A_PAD // 128, 128)
    out64 = jax.ShapeDtypeStruct((B, 64), jnp.float32)
    nms = pl.pallas_call(
        _nms_body,
        out_shape=[out64] * 6,
        scratch_shapes=[pltpu.VMEM((B, _A_PAD // 128, 128), jnp.float32)],
    )
    ss, cc, bx1, by1, bx2, by2 = nms(
        rs(s_w), rs(cls_f), rs(x1), rs(y1), rs(x2), rs(y2))

    boxes = jnp.stack([bx1[:, :_MAX_DET], by1[:, :_MAX_DET],
                       bx2[:, :_MAX_DET], by2[:, :_MAX_DET]], axis=-1)
    return ss[:, :_MAX_DET], cc[:, :_MAX_DET], boxes


# speedup vs baseline: 1.6134x; 1.0008x over previous
import functools

import jax
import jax.numpy as jnp
from jax import lax
from jax.experimental import pallas as pl
from jax.experimental.pallas import tpu as pltpu

_IMAGE_W = 1024
_IMAGE_H = 1024
_MIN_SCORE = 0.1
_MAX_DET = 50
_A = 20000
_A_PAD = 20480
_BLK = 2048
_NEG = float("-inf")


def _trunc(x):
    return jnp.where(x >= 0.0, jnp.floor(x), jnp.ceil(x))


def _prep_body(cls_ref, reg_ref, anc_ref,
               s_ref, c_ref, x1_ref, y1_ref, x2_ref, y2_ref):
    j = pl.program_id(1)
    lane = lax.broadcasted_iota(jnp.int32, (1, _BLK), 1)
    valid = (j * _BLK + lane) < _A
    c = cls_ref[0]
    s = jnp.max(c, axis=0, keepdims=True)
    iota_c = lax.broadcasted_iota(jnp.int32, c.shape, 0)
    cls_i = jnp.min(jnp.where(c == s, iota_c, c.shape[0]),
                    axis=0, keepdims=True)
    reg = reg_ref[0]
    anc = anc_ref[0]
    aw = anc[2:3] - anc[0:1]
    ah = anc[3:4] - anc[1:2]
    acx = anc[0:1] + 0.5 * aw
    acy = anc[1:2] + 0.5 * ah
    tx = reg[0:1] * 0.1
    ty = reg[1:2] * 0.1
    tw = reg[2:3] * 0.2
    th = reg[3:4] * 0.2
    w = jnp.exp(tw) * aw
    h = jnp.exp(th) * ah
    cx = tx * aw + acx
    cy = ty * ah + acy
    x1 = jnp.maximum(_trunc(cx - 0.5 * w), 0.0)
    y1 = jnp.maximum(_trunc(cy - 0.5 * h), 0.0)
    x2 = jnp.minimum(_trunc(cx + 0.5 * w), float(_IMAGE_W - 1))
    y2 = jnp.minimum(_trunc(cy + 0.5 * h), float(_IMAGE_H - 1))
    s_ref[0, 0] = jnp.where(valid & (s > _MIN_SCORE), s, _NEG)
    c_ref[0, 0] = jnp.where(valid, cls_i.astype(jnp.float32), 0.0)
    x1_ref[0, 0] = jnp.where(valid, x1, 0.0)
    y1_ref[0, 0] = jnp.where(valid, y1, 0.0)
    x2_ref[0, 0] = jnp.where(valid, x2, 0.0)
    y2_ref[0, 0] = jnp.where(valid, y2, 0.0)


def _nms_body(s_ref, c_ref, x1_ref, y1_ref, x2_ref, y2_ref,
              os_ref, oc_ref, ox1_ref, oy1_ref, ox2_ref, oy2_ref,
              sw_ref):
    B, R, L = s_ref.shape
    sw_ref[...] = s_ref[...]
    fill = jnp.full((B, 64), -1.0, jnp.float32)
    os_ref[...] = fill
    oc_ref[...] = fill
    ox1_ref[...] = fill
    oy1_ref[...] = fill
    ox2_ref[...] = fill
    oy2_ref[...] = fill
    lin = (lax.broadcasted_iota(jnp.int32, (R, L), 0) * L
           + lax.broadcasted_iota(jnp.int32, (R, L), 1))
    lane128 = lax.broadcasted_iota(jnp.int32, (1, L), 1)
    lane64 = lax.broadcasted_iota(jnp.int32, (1, 64), 1)

    def body(t, _):
        for b in range(B):
            sb = sw_ref[b]
            m = jnp.max(sb)
            has = m > _NEG
            pick = jnp.min(jnp.where(sb == m, lin, R * L))
            pr = jnp.minimum(pick // L, R - 1)
            pc = pick % L

            def gat(ref):
                row = ref[b, pl.ds(pr, 1), :]
                return jnp.sum(jnp.where(lane128 == pc, row, 0.0))

            px1 = gat(x1_ref)
            py1 = gat(y1_ref)
            px2 = gat(x2_ref)
            py2 = gat(y2_ref)
            pcl = gat(c_ref)
            par = (jnp.maximum(px2 - px1, 0.0)
                   * jnp.maximum(py2 - py1, 0.0))

            X1 = x1_ref[b]
            Y1 = y1_ref[b]
            X2 = x2_ref[b]
            Y2 = y2_ref[b]
            areas = jnp.maximum(X2 - X1, 0.0) * jnp.maximum(Y2 - Y1, 0.0)
            iw = jnp.maximum(jnp.minimum(px2, X2) - jnp.maximum(px1, X1), 0.0)
            ih = jnp.maximum(jnp.minimum(py2, Y2) - jnp.maximum(py1, Y1), 0.0)
            inter = iw * ih
            thr = 0.5 * (par + areas - inter + 1e-9)
            sup = (inter > thr) & (c_ref[b] == pcl)
            kill = (sup | (lin == pick)) & has
            sw_ref[b] = jnp.where(kill, _NEG, sb)

            oh = (lane64 == t) & has
            row_s = os_ref[pl.ds(b, 1), :]
            os_ref[pl.ds(b, 1), :] = jnp.where(oh, m, row_s)
            oc_ref[pl.ds(b, 1), :] = jnp.where(oh, pcl, oc_ref[pl.ds(b, 1), :])
            ox1_ref[pl.ds(b, 1), :] = jnp.where(oh, px1, ox1_ref[pl.ds(b, 1), :])
            oy1_ref[pl.ds(b, 1), :] = jnp.where(oh, py1, oy1_ref[pl.ds(b, 1), :])
            ox2_ref[pl.ds(b, 1), :] = jnp.where(oh, px2, ox2_ref[pl.ds(b, 1), :])
            oy2_ref[pl.ds(b, 1), :] = jnp.where(oh, py2, oy2_ref[pl.ds(b, 1), :])
        return 0

    lax.fori_loop(0, _MAX_DET, body, 0)


@jax.jit
def kernel(cls_heads, reg_heads, batch_anchors):
    B, A, C = cls_heads.shape
    cls_t = jnp.transpose(cls_heads, (0, 2, 1))
    reg_t = jnp.transpose(reg_heads, (0, 2, 1))
    anc_t = jnp.transpose(batch_anchors, (0, 2, 1))

    nblk = _A_PAD // _BLK
    flat = jax.ShapeDtypeStruct((B, nblk, 1, _BLK), jnp.float32)
    prep = pl.pallas_call(
        _prep_body,
        grid=(B, nblk),
        in_specs=[
            pl.BlockSpec((1, C, _BLK), lambda b, j: (b, 0, j)),
            pl.BlockSpec((1, 4, _BLK), lambda b, j: (b, 0, j)),
            pl.BlockSpec((1, 4, _BLK), lambda b, j: (b, 0, j)),
        ],
        out_specs=[pl.BlockSpec((1, 1, 1, _BLK), lambda b, j: (b, j, 0, 0))] * 6,
        out_shape=[flat] * 6,
    )
    s_w, cls_f, x1, y1, x2, y2 = prep(cls_t, reg_t, anc_t)

    rs = lambda a: a.reshape(B, _A_PAD // 128, 128)
    out64 = jax.ShapeDtypeStruct((B, 64), jnp.float32)
    nms = pl.pallas_call(
        _nms_body,
        out_shape=[out64] * 6,
        scratch_shapes=[pltpu.VMEM((B, _A_PAD // 128, 128), jnp.float32)],
    )
    ss, cc, bx1, by1, bx2, by2 = nms(
        rs(s_w), rs(cls_f), rs(x1), rs(y1), rs(x2), rs(y2))

    boxes = jnp.stack([bx1[:, :_MAX_DET], by1[:, :_MAX_DET],
                       bx2[:, :_MAX_DET], by2[:, :_MAX_DET]], axis=-1)
    return ss[:, :_MAX_DET], cc[:, :_MAX_DET], boxes
